# Initial kernel scaffold; baseline (speedup 1.0000x reference)
import jax
import jax.numpy as jnp
from jax.experimental import pallas as pl


def kernel(r_feat, u_feat, i_feat, y, W_agg, W_gcn, w_omega, b_omega, u_omega, u_cls, r2u, r2i, edge_index1, edge_index2, idx_mask):
    raise NotImplementedError("write your pallas kernel here")


# speedup vs baseline: 21.5142x; 21.5142x over previous
import functools

import jax
import jax.numpy as jnp
from jax import lax
from jax.experimental import pallas as pl
from jax.experimental.pallas import tpu as pltpu
from jax.experimental.pallas import tpu_sc as plsc

N_R = 10000
N_U = 5000
N_I = 5000
E = 320000
OUT = 128
CLS = 2
NC, NS, LANES = 2, 16, 16
NW = NC * NS

_f32 = jnp.float32


_MESH = dict(core_axis_name="c", subcore_axis_name="s", num_cores=NC,
             num_subcores=NS)

GB = 20480
GPT = GB // NW
GNR = GPT // 128


def _sc_gather(tab, idx3d):
    mesh = plsc.VectorSubcoreMesh(**_MESH)

    @functools.partial(
        pl.kernel, mesh=mesh,
        out_type=jax.ShapeDtypeStruct((GB, OUT), _f32),
        scratch_types=[
            pltpu.VMEM((GNR, 128), jnp.int32),
            pltpu.VMEM((128, OUT), _f32),
            pltpu.VMEM((128, OUT), _f32),
            pltpu.SemaphoreType.DMA,
            pltpu.SemaphoreType.DMA,
        ],
    )
    def k(tab_h, idx_h, out_h, idxv, rows0, rows1, sem0, sem1):
        wid = lax.axis_index("s") * NC + lax.axis_index("c")
        nrow = GNR
        pltpu.sync_copy(idx_h.at[wid], idxv)
        rows = (rows0, rows1)
        sems = (sem0, sem1)
        cps = [None, None]
        for j in range(nrow):
            b = j % 2
            cps[b] = pltpu.async_copy(tab_h.at[idxv.at[j]], rows[b], sems[b])
            if j > 0:
                cps[1 - b].wait()
                pltpu.sync_copy(rows[1 - b],
                                out_h.at[pl.ds(wid * GPT + (j - 1) * 128, 128)])
        cps[(nrow - 1) % 2].wait()
        pltpu.sync_copy(rows[(nrow - 1) % 2],
                        out_h.at[pl.ds(wid * GPT + (nrow - 1) * 128, 128)])

    return k(tab, idx3d)


DEG_PAD = 10240
NRP = 10240
EB = 125
EROWS = E // EB
ERPT = EROWS // NS
ECH = 32
MROWS = 32
MPT = 8
CW = 8


def _sc_counts(dst3d, mask3d, zbig, ones_e, ones_m):
    mesh = plsc.VectorSubcoreMesh(**_MESH)

    @functools.partial(
        pl.kernel, mesh=mesh,
        out_type=jax.ShapeDtypeStruct((2, 2, DEG_PAD, 128), _f32),
        scratch_types=[
            pltpu.VMEM_SHARED((DEG_PAD, 128), _f32),
            pltpu.VMEM((ECH, EB), jnp.int32),
            pltpu.VMEM((MPT, 128), jnp.int32),
            pltpu.VMEM((EB, 128), _f32),
            pltpu.VMEM((128, 128), _f32),
        ],
    )
    def k(dst_h, msk_h, z_h, onee_h, onem_h, out_h, cnt_sh, idxv, midxv,
          ones_ev, ones_mv):
        c = lax.axis_index("c")
        s = lax.axis_index("s")
        seg = DEG_PAD // NS
        pltpu.sync_copy(onee_h, ones_ev)
        pltpu.sync_copy(onem_h, ones_mv)
        pltpu.sync_copy(z_h.at[pl.ds(s * seg, seg)],
                        cnt_sh.at[pl.ds(s * seg, seg)])
        plsc.subcore_barrier()

        def chunk(ch, carry):
            pltpu.sync_copy(dst_h.at[c].at[pl.ds(s * ERPT + ch * ECH, ECH)],
                            idxv)
            for j in range(ECH):
                pltpu.sync_copy(ones_ev, cnt_sh.at[idxv.at[j]], add=True)
            return carry

        lax.fori_loop(0, ERPT // ECH, chunk, 0)
        plsc.subcore_barrier()
        pltpu.sync_copy(cnt_sh.at[pl.ds(s * seg, seg)],
                        out_h.at[c, 0, pl.ds(s * seg, seg)])
        pltpu.sync_copy(z_h.at[pl.ds(s * seg, seg)],
                        cnt_sh.at[pl.ds(s * seg, seg)])
        plsc.subcore_barrier()

        @pl.when(s < MROWS // MPT)
        def _():
            pltpu.sync_copy(msk_h.at[c].at[pl.ds(s * MPT, MPT)], midxv)
            for j in range(MPT):
                pltpu.sync_copy(ones_mv, cnt_sh.at[midxv.at[j]], add=True)

        plsc.subcore_barrier()
        pltpu.sync_copy(cnt_sh.at[pl.ds(s * seg, seg)],
                        out_h.at[c, 1, pl.ds(s * seg, seg)])

    return k(dst3d, mask3d, zbig, ones_e, ones_m)


def _sc_edge(pre2d, src3d, dst3d, zbig):
    mesh = plsc.VectorSubcoreMesh(**_MESH)

    @functools.partial(
        pl.kernel, mesh=mesh,
        out_type=jax.ShapeDtypeStruct((2, NRP, OUT), _f32),
        scratch_types=[
            pltpu.VMEM_SHARED((NRP, OUT), _f32),
            pltpu.VMEM((ECH, EB), jnp.int32),
            pltpu.VMEM((ECH, EB), jnp.int32),
            pltpu.VMEM((EB, OUT), _f32),
            pltpu.VMEM((EB, OUT), _f32),
            pltpu.SemaphoreType.DMA,
            pltpu.SemaphoreType.DMA,
        ],
    )
    def k(pre_h, src_h, dst_h, z_h, out_h, raw_sh, sidxv, didxv, rows0, rows1,
          sem0, sem1):
        c = lax.axis_index("c")
        s = lax.axis_index("s")
        seg = NRP // NS
        pltpu.sync_copy(z_h.at[pl.ds(s * seg, seg)], raw_sh.at[pl.ds(s * seg, seg)])
        plsc.subcore_barrier()
        rows = (rows0, rows1)
        sems = (sem0, sem1)

        def chunk(ch, carry):
            base = s * ERPT + ch * ECH
            pltpu.sync_copy(src_h.at[c].at[pl.ds(base, ECH)], sidxv)
            pltpu.sync_copy(dst_h.at[c].at[pl.ds(base, ECH)], didxv)
            cp0 = pltpu.async_copy(pre_h.at[sidxv.at[0]], rows[0], sems[0])
            for j in range(ECH):
                b = j % 2
                if j + 1 < ECH:
                    nxt = pltpu.async_copy(pre_h.at[sidxv.at[j + 1]],
                                           rows[1 - b], sems[1 - b])
                if j == 0:
                    cp0.wait()
                else:
                    prev.wait()
                pltpu.sync_copy(rows[b], raw_sh.at[didxv.at[j]], add=True)
                if j + 1 < ECH:
                    prev = nxt
            return carry

        lax.fori_loop(0, ERPT // ECH, chunk, 0)
        plsc.subcore_barrier()
        pltpu.sync_copy(raw_sh.at[pl.ds(s * seg, seg)],
                        out_h.at[c, pl.ds(s * seg, seg)])

    return k(pre2d, src3d, dst3d, zbig)


BLK = 1000
NBLK = N_R // BLK


def _tc_front(r_feat, gath, w0, w1, w2, wg, u1, deg1, deg2):
    def body(r_r, gu_r, gi_r, w0_r, w1_r, w2_r, wg_r, u1_r, d1_r, d2_r,
             ps_r, b0_r):
        h = jnp.maximum(
            r_r[...] @ w0_r[...] + gu_r[...] @ w1_r[...] + gi_r[...] @ w2_r[...],
            0.0)
        rs = jnp.sum(h, axis=1, keepdims=True)
        x = h * jnp.where(rs != 0.0, 1.0 / rs, 0.0)
        p = x @ wg_r[...]
        b0_r[...] = h @ u1_r[...]
        ps_r[0] = lax.rsqrt(d1_r[...] + 1.0) * p
        ps_r[1] = lax.rsqrt(d2_r[...] + 1.0) * p

    fspec = pl.BlockSpec((BLK, OUT), lambda i: (i, 0))
    return pl.pallas_call(
        body,
        grid=(NBLK,),
        in_specs=[
            fspec,
            pl.BlockSpec((BLK, OUT), lambda i: (i, 0)),
            pl.BlockSpec((BLK, OUT), lambda i: (i + NBLK, 0)),
            pl.BlockSpec((OUT, OUT), lambda i: (0, 0)),
            pl.BlockSpec((OUT, OUT), lambda i: (0, 0)),
            pl.BlockSpec((OUT, OUT), lambda i: (0, 0)),
            pl.BlockSpec((OUT, OUT), lambda i: (0, 0)),
            pl.BlockSpec((OUT, CLS), lambda i: (0, 0)),
            pl.BlockSpec((BLK, 1), lambda i: (i, 0)),
            pl.BlockSpec((BLK, 1), lambda i: (i, 0)),
        ],
        out_specs=[
            pl.BlockSpec((2, BLK, OUT), lambda i: (0, i, 0)),
            pl.BlockSpec((BLK, CLS), lambda i: (i, 0)),
        ],
        out_shape=[
            jax.ShapeDtypeStruct((2, N_R, OUT), _f32),
            jax.ShapeDtypeStruct((N_R, CLS), _f32),
        ],
    )(r_feat, gath, gath, w0, w1, w2, wg, u1, deg1, deg2)


def _tc_mid(raw, pre_s, deg1, deg2, w0m, w1m, u2):
    def body(raw_r, ps_r, d1_r, d2_r, w0_r, w1_r, u2_r, s_r, b1_r, b2_r):
        i = pl.program_id(0)

        @pl.when(i == 0)
        def _():
            s_r[0, 0] = 0.0
            s_r[0, 1] = 0.0
            s_r[1, 0] = 0.0
            s_r[1, 1] = 0.0

        w0b = w0_r[...]
        w1b = w1_r[...]
        for v, d_r, b_r, k in ((0, d1_r, b1_r, 0), (1, d2_r, b2_r, 1)):
            agg = lax.rsqrt(d_r[...] + 1.0) * (raw_r[v] + ps_r[v])
            nrm = lax.rsqrt(jnp.maximum(
                jnp.sum(agg * agg, axis=1, keepdims=True), 1e-12))
            g = jnp.maximum(agg * nrm, 0.0)
            s_r[k, 0] += jnp.sum(g * w0b)
            s_r[k, 1] += jnp.sum(g * w1b)
            b_r[...] = g @ u2_r[...]

    return pl.pallas_call(
        body,
        grid=(NBLK,),
        in_specs=[
            pl.BlockSpec((2, BLK, OUT), lambda i: (0, i, 0)),
            pl.BlockSpec((2, BLK, OUT), lambda i: (0, i, 0)),
            pl.BlockSpec((BLK, 1), lambda i: (i, 0)),
            pl.BlockSpec((BLK, 1), lambda i: (i, 0)),
            pl.BlockSpec((BLK, OUT), lambda i: (i, 0)),
            pl.BlockSpec((BLK, OUT), lambda i: (i, 0)),
            pl.BlockSpec((OUT, CLS), lambda i: (0, 0)),
        ],
        out_specs=[
            pl.BlockSpec(memory_space=pltpu.MemorySpace.SMEM),
            pl.BlockSpec((BLK, CLS), lambda i: (i, 0)),
            pl.BlockSpec((BLK, CLS), lambda i: (i, 0)),
        ],
        out_shape=[
            jax.ShapeDtypeStruct((2, 2), _f32),
            jax.ShapeDtypeStruct((N_R, CLS), _f32),
            jax.ShapeDtypeStruct((N_R, CLS), _f32),
        ],
    )(raw, pre_s, deg1, deg2, w0m, w1m, u2)


def _tc_loss(S, b_om, u_om, b0, b1, b2, y, cnts):
    def body(s_r, bo_r, uo_r, b0_r, b1_r, b2_r, y_r, cnt_r, out_r):
        i = pl.program_id(0)
        v00 = jnp.tanh(s_r[0, 0] + bo_r[0])
        v01 = jnp.tanh(s_r[0, 1] + bo_r[1])
        v10 = jnp.tanh(s_r[1, 0] + bo_r[0])
        v11 = jnp.tanh(s_r[1, 1] + bo_r[1])
        vu0 = v00 * uo_r[0] + v01 * uo_r[1]
        vu1 = v10 * uo_r[0] + v11 * uo_r[1]
        m = jnp.maximum(vu0, vu1)
        e0 = jnp.exp(vu0 - m)
        e1 = jnp.exp(vu1 - m)
        a0 = e0 / (e0 + e1)
        a1 = e1 / (e0 + e1)
        logit = b0_r[...] + a0 * b1_r[...] + a1 * b2_r[...]
        mx = jnp.max(logit, axis=1, keepdims=True)
        ex = jnp.exp(logit - mx)
        p = ex / jnp.sum(ex, axis=1, keepdims=True)
        t = jnp.log(1.0 / (1.0 + jnp.exp(-y_r[...] * p)))
        cnt = cnt_r[0] + cnt_r[1]
        blk = jnp.sum(cnt * t)

        @pl.when(i == 0)
        def _():
            out_r[0, 0] = 0.0

        out_r[0, 0] += -blk

    return pl.pallas_call(
        body,
        grid=(NBLK,),
        in_specs=[
            pl.BlockSpec(memory_space=pltpu.MemorySpace.SMEM),
            pl.BlockSpec(memory_space=pltpu.MemorySpace.SMEM),
            pl.BlockSpec(memory_space=pltpu.MemorySpace.SMEM),
            pl.BlockSpec((BLK, CLS), lambda i: (i, 0)),
            pl.BlockSpec((BLK, CLS), lambda i: (i, 0)),
            pl.BlockSpec((BLK, CLS), lambda i: (i, 0)),
            pl.BlockSpec((BLK, CLS), lambda i: (i, 0)),
            pl.BlockSpec((2, BLK, 1), lambda i: (0, i, 0)),
        ],
        out_specs=pl.BlockSpec(memory_space=pltpu.MemorySpace.SMEM),
        out_shape=jax.ShapeDtypeStruct((1, 1), _f32),
    )(S, b_om, u_om, b0, b1, b2, y, cnts)


def kernel(r_feat, u_feat, i_feat, y, W_agg, W_gcn, w_omega, b_omega, u_omega,
           u_cls, r2u, r2i, edge_index1, edge_index2, idx_mask):
    tab = jnp.concatenate([u_feat, i_feat], axis=0)
    idx_all = jnp.concatenate(
        [r2u, r2i + N_U, jnp.zeros((GB - 2 * N_R,), jnp.int32)]).reshape(
            NW, GNR, 128)
    dst3d = jnp.stack([edge_index1[1], edge_index2[1]]).reshape(2, EROWS, EB)
    src3d = jnp.stack([edge_index1[0], edge_index2[0] + N_R]).reshape(
        2, EROWS, EB)
    mask3d = jnp.concatenate(
        [idx_mask, jnp.full((8192 - idx_mask.shape[0],), N_R, jnp.int32)]
    ).reshape(2, MROWS, 128)
    ones_e = jnp.ones((EB, 128), _f32)
    ones_m = jnp.ones((128, 128), _f32)
    zbig = jnp.zeros((NRP, OUT), _f32)
    w0c, w1c, w2c = W_agg[:OUT], W_agg[OUT:2 * OUT], W_agg[2 * OUT:]
    u1, u2 = u_cls[:OUT], u_cls[OUT:]
    w0m = w_omega[:, 0].reshape(N_R, OUT)
    w1m = w_omega[:, 1].reshape(N_R, OUT)

    gath = _sc_gather(tab, idx_all)
    cnts = _sc_counts(dst3d, mask3d, zbig, ones_e, ones_m)
    deg1 = cnts[0, 0, :N_R, :1]
    deg2 = cnts[1, 0, :N_R, :1]
    mcnt = cnts[:, 1, :N_R, :1]
    pre_s, b0 = _tc_front(r_feat, gath, w0c, w1c, w2c, W_gcn, u1,
                          deg1, deg2)
    raw = _sc_edge(pre_s.reshape(2 * N_R, OUT), src3d, dst3d, zbig)
    S, b1, b2 = _tc_mid(raw, pre_s, deg1, deg2, w0m, w1m, u2)
    loss = _tc_loss(S, b_omega, u_omega, b0, b1, b2, y, mcnt)
    return loss.reshape(())


# speedup vs baseline: 22.8061x; 1.0600x over previous
import functools

import jax
import jax.numpy as jnp
from jax import lax
from jax.experimental import pallas as pl
from jax.experimental.pallas import tpu as pltpu
from jax.experimental.pallas import tpu_sc as plsc

N_R = 10000
N_U = 5000
N_I = 5000
E = 320000
OUT = 128
CLS = 2
NC, NS, LANES = 2, 16, 16
NW = NC * NS

_f32 = jnp.float32


_MESH = dict(core_axis_name="c", subcore_axis_name="s", num_cores=NC,
             num_subcores=NS)

GB = 20480
GPT = GB // NW
GNR = GPT // 128


DEG_PAD = 10240
NRP = 10240
EB = 125
EROWS = E // EB
ERPT = EROWS // NS
ECH = 32
MROWS = 32
MPT = 8
CW = 8


def _sc_front(tab, idx3d, dst3d, mask3d, zbig, ones_m):
    mesh = plsc.VectorSubcoreMesh(**_MESH)

    @functools.partial(
        pl.kernel, mesh=mesh,
        out_type=(jax.ShapeDtypeStruct((GB, OUT), _f32),
                  jax.ShapeDtypeStruct((2, 2, DEG_PAD, 128), _f32)),
        scratch_types=[
            pltpu.VMEM((GNR, 128), jnp.int32),
            pltpu.VMEM((128, OUT), _f32),
            pltpu.VMEM_SHARED((DEG_PAD, 128), _f32),
            pltpu.VMEM((ECH, EB), jnp.int32),
            pltpu.VMEM((MPT, 128), jnp.int32),
            pltpu.VMEM((128, 128), _f32),
            pltpu.SemaphoreType.DMA,
        ],
    )
    def k(tab_h, idx_h, dst_h, msk_h, z_h, onem_h, gout_h, cout_h,
          gidxv, grows, cnt_sh, idxv, midxv, ones_mv, gsem):
        c = lax.axis_index("c")
        s = lax.axis_index("s")
        wid = s * NC + c
        seg = DEG_PAD // NS

        pltpu.sync_copy(idx_h.at[wid], gidxv)
        gcp = pltpu.async_copy(tab_h.at[gidxv.at[0]], grows, gsem)

        pltpu.sync_copy(onem_h, ones_mv)
        pltpu.sync_copy(z_h.at[pl.ds(s * seg, seg)],
                        cnt_sh.at[pl.ds(s * seg, seg)])
        plsc.subcore_barrier()

        nch = ERPT // ECH
        for ch in range(nch):
            pltpu.sync_copy(dst_h.at[c].at[pl.ds(s * ERPT + ch * ECH, ECH)],
                            idxv)
            for j in range(ECH):
                pltpu.sync_copy(ones_mv.at[pl.ds(0, EB)],
                                cnt_sh.at[idxv.at[j]], add=True)
            if ch < GNR:
                gcp.wait()
                pltpu.sync_copy(grows,
                                gout_h.at[pl.ds(wid * GPT + ch * 128, 128)])
                if ch + 1 < GNR:
                    gcp = pltpu.async_copy(tab_h.at[gidxv.at[ch + 1]], grows,
                                           gsem)
        plsc.subcore_barrier()
        pltpu.sync_copy(cnt_sh.at[pl.ds(s * seg, seg)],
                        cout_h.at[c, 0, pl.ds(s * seg, seg)])
        pltpu.sync_copy(z_h.at[pl.ds(s * seg, seg)],
                        cnt_sh.at[pl.ds(s * seg, seg)])
        plsc.subcore_barrier()

        @pl.when(s < MROWS // MPT)
        def _():
            pltpu.sync_copy(msk_h.at[c].at[pl.ds(s * MPT, MPT)], midxv)
            for j in range(MPT):
                pltpu.sync_copy(ones_mv, cnt_sh.at[midxv.at[j]], add=True)

        plsc.subcore_barrier()
        pltpu.sync_copy(cnt_sh.at[pl.ds(s * seg, seg)],
                        cout_h.at[c, 1, pl.ds(s * seg, seg)])

    return k(tab, idx3d, dst3d, mask3d, zbig, ones_m)


def _sc_edge(pre2d, src3d, dst3d, zbig):
    mesh = plsc.VectorSubcoreMesh(**_MESH)

    @functools.partial(
        pl.kernel, mesh=mesh,
        out_type=jax.ShapeDtypeStruct((2, NRP, OUT), _f32),
        scratch_types=[
            pltpu.VMEM_SHARED((NRP, OUT), _f32),
            pltpu.VMEM((ECH, EB), jnp.int32),
            pltpu.VMEM((ECH, EB), jnp.int32),
            pltpu.VMEM((EB, OUT), _f32),
            pltpu.VMEM((EB, OUT), _f32),
            pltpu.SemaphoreType.DMA,
            pltpu.SemaphoreType.DMA,
        ],
    )
    def k(pre_h, src_h, dst_h, z_h, out_h, raw_sh, sidxv, didxv, rows0, rows1,
          sem0, sem1):
        c = lax.axis_index("c")
        s = lax.axis_index("s")
        seg = NRP // NS
        pltpu.sync_copy(z_h.at[pl.ds(s * seg, seg)], raw_sh.at[pl.ds(s * seg, seg)])
        plsc.subcore_barrier()
        rows = (rows0, rows1)
        sems = (sem0, sem1)

        def chunk(ch, carry):
            base = s * ERPT + ch * ECH
            pltpu.sync_copy(src_h.at[c].at[pl.ds(base, ECH)], sidxv)
            pltpu.sync_copy(dst_h.at[c].at[pl.ds(base, ECH)], didxv)
            cp0 = pltpu.async_copy(pre_h.at[sidxv.at[0]], rows[0], sems[0])
            for j in range(ECH):
                b = j % 2
                if j + 1 < ECH:
                    nxt = pltpu.async_copy(pre_h.at[sidxv.at[j + 1]],
                                           rows[1 - b], sems[1 - b])
                if j == 0:
                    cp0.wait()
                else:
                    prev.wait()
                pltpu.sync_copy(rows[b], raw_sh.at[didxv.at[j]], add=True)
                if j + 1 < ECH:
                    prev = nxt
            return carry

        lax.fori_loop(0, ERPT // ECH, chunk, 0)
        plsc.subcore_barrier()
        pltpu.sync_copy(raw_sh.at[pl.ds(s * seg, seg)],
                        out_h.at[c, pl.ds(s * seg, seg)])

    return k(pre2d, src3d, dst3d, zbig)


BLK = 1000
NBLK = N_R // BLK


def _tc_front(r_feat, gath, w0, w1, w2, wg, u1, deg1, deg2):
    def body(r_r, gu_r, gi_r, w0_r, w1_r, w2_r, wg_r, u1_r, d1_r, d2_r,
             ps_r, b0_r):
        h = jnp.maximum(
            r_r[...] @ w0_r[...] + gu_r[...] @ w1_r[...] + gi_r[...] @ w2_r[...],
            0.0)
        rs = jnp.sum(h, axis=1, keepdims=True)
        x = h * jnp.where(rs != 0.0, 1.0 / rs, 0.0)
        p = x @ wg_r[...]
        b0_r[...] = h @ u1_r[...]
        ps_r[0] = lax.rsqrt(d1_r[...] + 1.0) * p
        ps_r[1] = lax.rsqrt(d2_r[...] + 1.0) * p

    fspec = pl.BlockSpec((BLK, OUT), lambda i: (i, 0))
    return pl.pallas_call(
        body,
        grid=(NBLK,),
        in_specs=[
            fspec,
            pl.BlockSpec((BLK, OUT), lambda i: (i, 0)),
            pl.BlockSpec((BLK, OUT), lambda i: (i + NBLK, 0)),
            pl.BlockSpec((OUT, OUT), lambda i: (0, 0)),
            pl.BlockSpec((OUT, OUT), lambda i: (0, 0)),
            pl.BlockSpec((OUT, OUT), lambda i: (0, 0)),
            pl.BlockSpec((OUT, OUT), lambda i: (0, 0)),
            pl.BlockSpec((OUT, CLS), lambda i: (0, 0)),
            pl.BlockSpec((BLK, 1), lambda i: (i, 0)),
            pl.BlockSpec((BLK, 1), lambda i: (i, 0)),
        ],
        out_specs=[
            pl.BlockSpec((2, BLK, OUT), lambda i: (0, i, 0)),
            pl.BlockSpec((BLK, CLS), lambda i: (i, 0)),
        ],
        out_shape=[
            jax.ShapeDtypeStruct((2, N_R, OUT), _f32),
            jax.ShapeDtypeStruct((N_R, CLS), _f32),
        ],
    )(r_feat, gath, gath, w0, w1, w2, wg, u1, deg1, deg2)


def _tc_mid(raw, pre_s, deg1, deg2, w0m, w1m, u2):
    def body(raw_r, ps_r, d1_r, d2_r, w0_r, w1_r, u2_r, s_r, b1_r, b2_r):
        i = pl.program_id(0)

        @pl.when(i == 0)
        def _():
            s_r[0, 0] = 0.0
            s_r[0, 1] = 0.0
            s_r[1, 0] = 0.0
            s_r[1, 1] = 0.0

        w0b = w0_r[...]
        w1b = w1_r[...]
        for v, d_r, b_r, k in ((0, d1_r, b1_r, 0), (1, d2_r, b2_r, 1)):
            agg = lax.rsqrt(d_r[...] + 1.0) * (raw_r[v] + ps_r[v])
            nrm = lax.rsqrt(jnp.maximum(
                jnp.sum(agg * agg, axis=1, keepdims=True), 1e-12))
            g = jnp.maximum(agg * nrm, 0.0)
            s_r[k, 0] += jnp.sum(g * w0b)
            s_r[k, 1] += jnp.sum(g * w1b)
            b_r[...] = g @ u2_r[...]

    return pl.pallas_call(
        body,
        grid=(NBLK,),
        in_specs=[
            pl.BlockSpec((2, BLK, OUT), lambda i: (0, i, 0)),
            pl.BlockSpec((2, BLK, OUT), lambda i: (0, i, 0)),
            pl.BlockSpec((BLK, 1), lambda i: (i, 0)),
            pl.BlockSpec((BLK, 1), lambda i: (i, 0)),
            pl.BlockSpec((BLK, OUT), lambda i: (i, 0)),
            pl.BlockSpec((BLK, OUT), lambda i: (i, 0)),
            pl.BlockSpec((OUT, CLS), lambda i: (0, 0)),
        ],
        out_specs=[
            pl.BlockSpec(memory_space=pltpu.MemorySpace.SMEM),
            pl.BlockSpec((BLK, CLS), lambda i: (i, 0)),
            pl.BlockSpec((BLK, CLS), lambda i: (i, 0)),
        ],
        out_shape=[
            jax.ShapeDtypeStruct((2, 2), _f32),
            jax.ShapeDtypeStruct((N_R, CLS), _f32),
            jax.ShapeDtypeStruct((N_R, CLS), _f32),
        ],
    )(raw, pre_s, deg1, deg2, w0m, w1m, u2)


def _tc_loss(S, b_om, u_om, b0, b1, b2, y, cnts):
    def body(s_r, bo_r, uo_r, b0_r, b1_r, b2_r, y_r, cnt_r, out_r):
        i = pl.program_id(0)
        v00 = jnp.tanh(s_r[0, 0] + bo_r[0])
        v01 = jnp.tanh(s_r[0, 1] + bo_r[1])
        v10 = jnp.tanh(s_r[1, 0] + bo_r[0])
        v11 = jnp.tanh(s_r[1, 1] + bo_r[1])
        vu0 = v00 * uo_r[0] + v01 * uo_r[1]
        vu1 = v10 * uo_r[0] + v11 * uo_r[1]
        m = jnp.maximum(vu0, vu1)
        e0 = jnp.exp(vu0 - m)
        e1 = jnp.exp(vu1 - m)
        a0 = e0 / (e0 + e1)
        a1 = e1 / (e0 + e1)
        logit = b0_r[...] + a0 * b1_r[...] + a1 * b2_r[...]
        mx = jnp.max(logit, axis=1, keepdims=True)
        ex = jnp.exp(logit - mx)
        p = ex / jnp.sum(ex, axis=1, keepdims=True)
        t = jnp.log(1.0 / (1.0 + jnp.exp(-y_r[...] * p)))
        cnt = cnt_r[0] + cnt_r[1]
        blk = jnp.sum(cnt * t)

        @pl.when(i == 0)
        def _():
            out_r[0, 0] = 0.0

        out_r[0, 0] += -blk

    return pl.pallas_call(
        body,
        grid=(NBLK,),
        in_specs=[
            pl.BlockSpec(memory_space=pltpu.MemorySpace.SMEM),
            pl.BlockSpec(memory_space=pltpu.MemorySpace.SMEM),
            pl.BlockSpec(memory_space=pltpu.MemorySpace.SMEM),
            pl.BlockSpec((BLK, CLS), lambda i: (i, 0)),
            pl.BlockSpec((BLK, CLS), lambda i: (i, 0)),
            pl.BlockSpec((BLK, CLS), lambda i: (i, 0)),
            pl.BlockSpec((BLK, CLS), lambda i: (i, 0)),
            pl.BlockSpec((2, BLK, 1), lambda i: (0, i, 0)),
        ],
        out_specs=pl.BlockSpec(memory_space=pltpu.MemorySpace.SMEM),
        out_shape=jax.ShapeDtypeStruct((1, 1), _f32),
    )(S, b_om, u_om, b0, b1, b2, y, cnts)


def kernel(r_feat, u_feat, i_feat, y, W_agg, W_gcn, w_omega, b_omega, u_omega,
           u_cls, r2u, r2i, edge_index1, edge_index2, idx_mask):
    tab = jnp.concatenate([u_feat, i_feat], axis=0)
    idx_all = jnp.concatenate(
        [r2u, r2i + N_U, jnp.zeros((GB - 2 * N_R,), jnp.int32)]).reshape(
            NW, GNR, 128)
    dst3d = jnp.stack([edge_index1[1], edge_index2[1]]).reshape(2, EROWS, EB)
    src3d = jnp.stack([edge_index1[0], edge_index2[0] + N_R]).reshape(
        2, EROWS, EB)
    mask3d = jnp.concatenate(
        [idx_mask, jnp.full((8192 - idx_mask.shape[0],), N_R, jnp.int32)]
    ).reshape(2, MROWS, 128)
    ones_m = jnp.ones((128, 128), _f32)
    zbig = jnp.zeros((NRP, OUT), _f32)
    w0c, w1c, w2c = W_agg[:OUT], W_agg[OUT:2 * OUT], W_agg[2 * OUT:]
    u1, u2 = u_cls[:OUT], u_cls[OUT:]
    w0m = w_omega[:, 0].reshape(N_R, OUT)
    w1m = w_omega[:, 1].reshape(N_R, OUT)

    gath, cnts = _sc_front(tab, idx_all, dst3d, mask3d, zbig,
                           ones_m)
    deg1 = cnts[0, 0, :N_R, :1]
    deg2 = cnts[1, 0, :N_R, :1]
    mcnt = cnts[:, 1, :N_R, :1]
    pre_s, b0 = _tc_front(r_feat, gath, w0c, w1c, w2c, W_gcn, u1,
                          deg1, deg2)
    raw = _sc_edge(pre_s.reshape(2 * N_R, OUT), src3d, dst3d, zbig)
    S, b1, b2 = _tc_mid(raw, pre_s, deg1, deg2, w0m, w1m, u2)
    loss = _tc_loss(S, b_omega, u_omega, b0, b1, b2, y, mcnt)
    return loss.reshape(())
